# Initial kernel scaffold; baseline (speedup 1.0000x reference)
import jax
import jax.numpy as jnp
from jax.experimental import pallas as pl


def kernel(objects):
    raise NotImplementedError("write your pallas kernel here")


# speedup vs baseline: 3.4359x; 3.4359x over previous
import functools

import jax
import jax.numpy as jnp
from jax import lax
from jax.experimental import pallas as pl
from jax.experimental.pallas import tpu as pltpu
from jax.experimental.pallas import tpu_sc as plsc

BATCH = 32
NOBJ = 128
GRID = 128
NCH = 84
CC = 3
CHUNK = CC * GRID * GRID
NCHUNK = NCH // CC
SLAB = NCH * GRID * GRID
NLANE = 16
FARIDX = 1 << 27


def kernel(objects):
    obj_t = objects.transpose(0, 2, 1)

    mesh = plsc.VectorSubcoreMesh(core_axis_name="c", subcore_axis_name="s")

    @functools.partial(
        pl.kernel,
        out_type=jax.ShapeDtypeStruct((BATCH, SLAB), jnp.float32),
        mesh=mesh,
        compiler_params=pltpu.CompilerParams(needs_layout_passes=False),
        scratch_types=[
            pltpu.VMEM((6 * NOBJ,), jnp.float32),
            pltpu.VMEM((NOBJ * NLANE,), jnp.int32),
            pltpu.VMEM((NOBJ * NLANE,), jnp.float32),
            pltpu.VMEM((CHUNK,), jnp.float32),
            pltpu.VMEM((CHUNK,), jnp.float32),
            pltpu.SemaphoreType.DMA,
            pltpu.SemaphoreType.DMA,
        ],
    )
    def scatter_kernel(obj_hbm, out_hbm, obj_v, idx_v, val_v, buf_a, buf_b,
                       sem_a, sem_b):
        b = lax.axis_index("c") * 16 + lax.axis_index("s")
        for r in range(6):
            pltpu.sync_copy(obj_hbm.at[b, r], obj_v.at[pl.ds(r * NOBJ, NOBJ)])

        lane = lax.iota(jnp.int32, 16)
        is_tap = lane < 9
        third = jnp.where(lane >= 3, 1, 0) + jnp.where(lane >= 6, 1, 0)
        dy = jnp.where(is_tap, third - 1, 0)
        dx = jnp.where(is_tap, lane - 3 * third - 1, 0)
        w_tap = jnp.exp(-1.125 * (dy * dy + dx * dx).astype(jnp.float32))
        reg_ch = 80 + (lane - 9)
        zeros_f = jnp.zeros((NLANE,), jnp.float32)

        def build(n, carry):
            nv = jnp.full((NLANE,), n, jnp.int32)
            y = plsc.load_gather(obj_v, [nv])
            x = plsc.load_gather(obj_v, [nv + NOBJ])
            hh = plsc.load_gather(obj_v, [nv + 2 * NOBJ])
            ww = plsc.load_gather(obj_v, [nv + 3 * NOBJ])
            cl = plsc.load_gather(obj_v, [nv + 4 * NOBJ])
            cf = plsc.load_gather(obj_v, [nv + 5 * NOBJ])
            yi = y.astype(jnp.int32)
            xi = x.astype(jnp.int32)
            ci = cl.astype(jnp.int32)
            oy = y - yi.astype(jnp.float32)
            ox = x - xi.astype(jnp.float32)
            ch = jnp.where(is_tap, ci, reg_ch)
            yy = yi + dy
            xx = xi + dx
            live = (lane < 13) & (cf == 1.0)
            inb = (yy >= 0) & (yy < GRID) & (xx >= 0) & (xx < GRID)
            m = live & inb
            idx = (ch << 14) + (yy << 7) + xx
            idx = jnp.where(m, idx, FARIDX)
            val = jnp.where(is_tap, w_tap, 0.0)
            val = jnp.where(lane == 9, oy, val)
            val = jnp.where(lane == 10, ox, val)
            val = jnp.where(lane == 11, hh, val)
            val = jnp.where(lane == 12, ww, val)
            idx_v[pl.ds(n * NLANE, NLANE)] = idx
            val_v[pl.ds(n * NLANE, NLANE)] = val
            return carry

        lax.fori_loop(0, NOBJ, build, 0)

        def zero_bufs(i, carry):
            for t in range(4):
                off = (i * 4 + t) * NLANE
                buf_a[pl.ds(off, NLANE)] = zeros_f
                buf_b[pl.ds(off, NLANE)] = zeros_f
            return carry

        lax.fori_loop(0, CHUNK // (4 * NLANE), zero_bufs, 0)

        def scan(buf, lo, add):
            hi = lo + CHUNK

            def body(n, carry):
                for t in range(4):
                    base = (n * 4 + t) * NLANE
                    iv = idx_v[pl.ds(base, NLANE)]
                    m = (iv >= lo) & (iv < hi)
                    li = iv - lo
                    if add:
                        vv = val_v[pl.ds(base, NLANE)]
                        plsc.addupdate_scatter(buf, [li], vv, mask=m)
                    else:
                        plsc.store_scatter(buf, [li], zeros_f, mask=m)
                return carry

            lax.fori_loop(0, NOBJ // 4, body, 0)

        def out_chunk(k):
            return out_hbm.at[b, pl.ds(k * CHUNK, CHUNK)]

        def pair(i, carry):
            k0 = 2 * i

            @pl.when(i > 0)
            def _():
                pltpu.make_async_copy(buf_a, out_chunk(k0 - 2), sem_a).wait()
                scan(buf_a, (k0 - 2) * CHUNK, add=False)

            scan(buf_a, k0 * CHUNK, add=True)
            pltpu.make_async_copy(buf_a, out_chunk(k0), sem_a).start()

            @pl.when(i > 0)
            def _():
                pltpu.make_async_copy(buf_b, out_chunk(k0 - 1), sem_b).wait()
                scan(buf_b, (k0 - 1) * CHUNK, add=False)

            scan(buf_b, (k0 + 1) * CHUNK, add=True)
            pltpu.make_async_copy(buf_b, out_chunk(k0 + 1), sem_b).start()
            return carry

        lax.fori_loop(0, NCHUNK // 2, pair, 0)
        pltpu.make_async_copy(buf_a, out_chunk(NCHUNK - 2), sem_a).wait()
        pltpu.make_async_copy(buf_b, out_chunk(NCHUNK - 1), sem_b).wait()

    out = scatter_kernel(obj_t)
    return out.reshape(BATCH, NCH, GRID, GRID)


# speedup vs baseline: 7.8360x; 2.2806x over previous
import functools

import jax
import jax.numpy as jnp
from jax import lax
from jax.experimental import pallas as pl
from jax.experimental.pallas import tpu as pltpu
from jax.experimental.pallas import tpu_sc as plsc

BATCH = 32
NOBJ = 128
GRID = 128
NCH = 84
CC = 3
NCHUNK = NCH // CC
NLANE = 16
FARCH = 1 << 20


def kernel(objects):
    obj_t = objects.transpose(0, 2, 1)

    mesh = plsc.VectorSubcoreMesh(core_axis_name="c", subcore_axis_name="s")

    @functools.partial(
        pl.kernel,
        out_type=jax.ShapeDtypeStruct((BATCH, NCH, GRID, GRID), jnp.float32),
        mesh=mesh,
        compiler_params=pltpu.CompilerParams(needs_layout_passes=False),
        scratch_types=[
            pltpu.VMEM((6 * NOBJ,), jnp.float32),
            pltpu.VMEM((NOBJ * NLANE,), jnp.int32),
            pltpu.VMEM((NOBJ * NLANE,), jnp.int32),
            pltpu.VMEM((NOBJ * NLANE,), jnp.int32),
            pltpu.VMEM((NOBJ * NLANE,), jnp.float32),
            pltpu.VMEM((CC, GRID, GRID), jnp.float32),
            pltpu.VMEM((CC, GRID, GRID), jnp.float32),
            pltpu.SemaphoreType.DMA,
            pltpu.SemaphoreType.DMA,
        ],
    )
    def scatter_kernel(obj_hbm, out_hbm, obj_v, ch_v, y_v, x_v, val_v,
                       buf_a, buf_b, sem_a, sem_b):
        b = lax.axis_index("c") * 16 + lax.axis_index("s")
        for r in range(6):
            pltpu.sync_copy(obj_hbm.at[b, r], obj_v.at[pl.ds(r * NOBJ, NOBJ)])

        lane = lax.iota(jnp.int32, 16)
        is_tap = lane < 9
        third = jnp.where(lane >= 3, 1, 0) + jnp.where(lane >= 6, 1, 0)
        dy = jnp.where(is_tap, third - 1, 0)
        dx = jnp.where(is_tap, lane - 3 * third - 1, 0)
        w_tap = jnp.exp(-1.125 * (dy * dy + dx * dx).astype(jnp.float32))
        reg_ch = 80 + (lane - 9)
        zeros_f = jnp.zeros((NLANE,), jnp.float32)

        def build(n, carry):
            nv = jnp.full((NLANE,), n, jnp.int32)
            y = plsc.load_gather(obj_v, [nv])
            x = plsc.load_gather(obj_v, [nv + NOBJ])
            hh = plsc.load_gather(obj_v, [nv + 2 * NOBJ])
            ww = plsc.load_gather(obj_v, [nv + 3 * NOBJ])
            cl = plsc.load_gather(obj_v, [nv + 4 * NOBJ])
            cf = plsc.load_gather(obj_v, [nv + 5 * NOBJ])
            yi = y.astype(jnp.int32)
            xi = x.astype(jnp.int32)
            ci = cl.astype(jnp.int32)
            oy = y - yi.astype(jnp.float32)
            ox = x - xi.astype(jnp.float32)
            ch = jnp.where(is_tap, ci, reg_ch)
            yy = yi + dy
            xx = xi + dx
            live = (lane < 13) & (cf == 1.0)
            inb = (yy >= 0) & (yy < GRID) & (xx >= 0) & (xx < GRID)
            m = live & inb
            ch = jnp.where(m, ch, FARCH)
            val = jnp.where(is_tap, w_tap, 0.0)
            val = jnp.where(lane == 9, oy, val)
            val = jnp.where(lane == 10, ox, val)
            val = jnp.where(lane == 11, hh, val)
            val = jnp.where(lane == 12, ww, val)
            ch_v[pl.ds(n * NLANE, NLANE)] = ch
            y_v[pl.ds(n * NLANE, NLANE)] = yy
            x_v[pl.ds(n * NLANE, NLANE)] = xx
            val_v[pl.ds(n * NLANE, NLANE)] = val
            return carry

        lax.fori_loop(0, NOBJ, build, 0)

        def zero_bufs(yrow, carry):
            for c in range(CC):
                for xb in range(GRID // NLANE):
                    buf_a[c, yrow, pl.ds(xb * NLANE, NLANE)] = zeros_f
                    buf_b[c, yrow, pl.ds(xb * NLANE, NLANE)] = zeros_f
            return carry

        lax.fori_loop(0, GRID, zero_bufs, 0)

        def scan(buf, c0, add):
            def body(n, carry):
                for t in range(4):
                    base = (n * 4 + t) * NLANE
                    cv = ch_v[pl.ds(base, NLANE)] - c0
                    yv = y_v[pl.ds(base, NLANE)]
                    xv = x_v[pl.ds(base, NLANE)]
                    m = (cv >= 0) & (cv < CC)
                    if add:
                        vv = val_v[pl.ds(base, NLANE)]
                        plsc.addupdate_scatter(buf, [cv, yv, xv], vv, mask=m)
                    else:
                        plsc.store_scatter(buf, [cv, yv, xv], zeros_f, mask=m)
                return carry

            lax.fori_loop(0, NOBJ // 4, body, 0)

        def out_chunk(k):
            return out_hbm.at[b, pl.ds(k * CC, CC)]

        def pair(i, carry):
            k0 = 2 * i

            @pl.when(i > 0)
            def _():
                pltpu.make_async_copy(buf_a, out_chunk(k0 - 2), sem_a).wait()
                scan(buf_a, (k0 - 2) * CC, add=False)

            scan(buf_a, k0 * CC, add=True)
            pltpu.make_async_copy(buf_a, out_chunk(k0), sem_a).start()

            @pl.when(i > 0)
            def _():
                pltpu.make_async_copy(buf_b, out_chunk(k0 - 1), sem_b).wait()
                scan(buf_b, (k0 - 1) * CC, add=False)

            scan(buf_b, (k0 + 1) * CC, add=True)
            pltpu.make_async_copy(buf_b, out_chunk(k0 + 1), sem_b).start()
            return carry

        lax.fori_loop(0, NCHUNK // 2, pair, 0)
        pltpu.make_async_copy(buf_a, out_chunk(NCHUNK - 2), sem_a).wait()
        pltpu.make_async_copy(buf_b, out_chunk(NCHUNK - 1), sem_b).wait()

    return scatter_kernel(obj_t)


# speedup vs baseline: 8.6732x; 1.1068x over previous
import functools

import jax
import jax.numpy as jnp
from jax import lax
from jax.experimental import pallas as pl
from jax.experimental.pallas import tpu as pltpu
from jax.experimental.pallas import tpu_sc as plsc

BATCH = 32
NOBJ = 128
GRID = 128
NCH = 84
CC = 3
NCHUNK = NCH // CC
NLANE = 16
FARIDX = 1 << 27


def kernel(objects):
    obj_t = objects.transpose(0, 2, 1)

    mesh = plsc.VectorSubcoreMesh(core_axis_name="c", subcore_axis_name="s")

    @functools.partial(
        pl.kernel,
        out_type=jax.ShapeDtypeStruct((BATCH, NCH, GRID, GRID), jnp.float32),
        mesh=mesh,
        compiler_params=pltpu.CompilerParams(needs_layout_passes=False),
        scratch_types=[
            pltpu.VMEM((6 * NOBJ,), jnp.float32),
            pltpu.VMEM((NOBJ * NLANE,), jnp.int32),
            pltpu.VMEM((NOBJ * NLANE,), jnp.float32),
            pltpu.VMEM((NOBJ * NLANE,), jnp.int32),
            pltpu.VMEM((NOBJ * NLANE,), jnp.float32),
            pltpu.VMEM((NCHUNK * NOBJ,), jnp.int32),
            pltpu.VMEM((32,), jnp.int32),
            pltpu.VMEM((CC, GRID, GRID), jnp.float32),
            pltpu.VMEM((CC, GRID, GRID), jnp.float32),
            pltpu.SemaphoreType.DMA,
            pltpu.SemaphoreType.DMA,
        ],
    )
    def scatter_kernel(obj_hbm, out_hbm, obj_v, gi_v, gv_v, ri_v, rv_v,
                       bkt_v, cnt_v, buf_a, buf_b, sem_a, sem_b):
        b = lax.axis_index("c") * 16 + lax.axis_index("s")
        for r in range(6):
            pltpu.sync_copy(obj_hbm.at[b, r], obj_v.at[pl.ds(r * NOBJ, NOBJ)])

        lane = lax.iota(jnp.int32, 16)
        is_tap = lane < 9
        is_reg = (lane >= 9) & (lane < 13)
        third = jnp.where(lane >= 3, 1, 0) + jnp.where(lane >= 6, 1, 0)
        dy = jnp.where(is_tap, third - 1, 0)
        dx = jnp.where(is_tap, lane - 3 * third - 1, 0)
        w_tap = jnp.exp(-1.125 * (dy * dy + dx * dx).astype(jnp.float32))
        reg_ch = 80 + (lane - 9)
        zeros_f = jnp.zeros((NLANE,), jnp.float32)
        lane0 = lane == 0

        cnt_v[pl.ds(0, NLANE)] = jnp.zeros((NLANE,), jnp.int32)
        cnt_v[pl.ds(NLANE, NLANE)] = jnp.zeros((NLANE,), jnp.int32)

        def build(n, carry):
            nv = jnp.full((NLANE,), n, jnp.int32)
            y = plsc.load_gather(obj_v, [nv])
            x = plsc.load_gather(obj_v, [nv + NOBJ])
            hh = plsc.load_gather(obj_v, [nv + 2 * NOBJ])
            ww = plsc.load_gather(obj_v, [nv + 3 * NOBJ])
            cl = plsc.load_gather(obj_v, [nv + 4 * NOBJ])
            cf = plsc.load_gather(obj_v, [nv + 5 * NOBJ])
            yi = y.astype(jnp.int32)
            xi = x.astype(jnp.int32)
            ci = cl.astype(jnp.int32)
            oy = y - yi.astype(jnp.float32)
            ox = x - xi.astype(jnp.float32)
            ch = jnp.where(is_tap, ci, reg_ch)
            yy = yi + dy
            xx = xi + dx
            real = cf == 1.0
            inb = (yy >= 0) & (yy < GRID) & (xx >= 0) & (xx < GRID)
            m = real & inb
            idx = (ch << 14) + (yy << 7) + xx
            gi_v[pl.ds(n * NLANE, NLANE)] = jnp.where(
                m & is_tap, idx, FARIDX)
            gv_v[pl.ds(n * NLANE, NLANE)] = w_tap
            ri_v[pl.ds(n * NLANE, NLANE)] = jnp.where(
                m & is_reg, idx, FARIDX)
            val = jnp.where(lane == 9, oy, 0.0)
            val = jnp.where(lane == 10, ox, val)
            val = jnp.where(lane == 11, hh, val)
            val = jnp.where(lane == 12, ww, val)
            rv_v[pl.ds(n * NLANE, NLANE)] = val
            cid = (ci * 21846) >> 16
            cnt = plsc.load_gather(cnt_v, [cid])
            mask0 = lane0 & real
            plsc.store_scatter(bkt_v, [cid * NOBJ + cnt], nv, mask=mask0)
            plsc.store_scatter(cnt_v, [cid], cnt + 1, mask=mask0)
            return carry

        lax.fori_loop(0, NOBJ, build, 0)

        def zero_bufs(yrow, carry):
            for c in range(CC):
                for xb in range(GRID // NLANE):
                    buf_a[c, yrow, pl.ds(xb * NLANE, NLANE)] = zeros_f
                    buf_b[c, yrow, pl.ds(xb * NLANE, NLANE)] = zeros_f
            return carry

        lax.fori_loop(0, GRID, zero_bufs, 0)

        def gauss_scan(buf, k, add):
            kv = jnp.full((NLANE,), k, jnp.int32)
            cntk = jnp.max(plsc.load_gather(cnt_v, [kv]))
            c0 = k * CC

            def gbody(j, carry):
                nvec = plsc.load_gather(bkt_v, [kv * NOBJ + j])
                iv = plsc.load_gather(gi_v, [nvec * NLANE + lane])
                cv = (iv >> 14) - c0
                yv = (iv >> 7) & (GRID - 1)
                xv = iv & (GRID - 1)
                m = (cv >= 0) & (cv < CC)
                if add:
                    vv = plsc.load_gather(gv_v, [nvec * NLANE + lane])
                    plsc.addupdate_scatter(buf, [cv, yv, xv], vv, mask=m)
                else:
                    plsc.store_scatter(buf, [cv, yv, xv], zeros_f, mask=m)
                return carry

            lax.fori_loop(0, cntk, gbody, 0)

        def reg_scan(buf, c0, add):
            def rbody(n, carry):
                for t in range(4):
                    base = (n * 4 + t) * NLANE
                    iv = ri_v[pl.ds(base, NLANE)]
                    cv = (iv >> 14) - c0
                    yv = (iv >> 7) & (GRID - 1)
                    xv = iv & (GRID - 1)
                    m = (cv >= 0) & (cv < CC)
                    if add:
                        vv = rv_v[pl.ds(base, NLANE)]
                        plsc.addupdate_scatter(buf, [cv, yv, xv], vv, mask=m)
                    else:
                        plsc.store_scatter(buf, [cv, yv, xv], zeros_f, mask=m)
                return carry

            lax.fori_loop(0, NOBJ // 4, rbody, 0)

        def out_chunk(k):
            return out_hbm.at[b, pl.ds(k * CC, CC)]

        def pair(i, carry):
            k0 = 2 * i

            @pl.when(i > 0)
            def _():
                pltpu.make_async_copy(buf_a, out_chunk(k0 - 2), sem_a).wait()
                gauss_scan(buf_a, k0 - 2, add=False)

            gauss_scan(buf_a, k0, add=True)

            @pl.when(k0 == NCHUNK - 2)
            def _():
                reg_scan(buf_a, (NCHUNK - 2) * CC, add=True)

            pltpu.make_async_copy(buf_a, out_chunk(k0), sem_a).start()

            @pl.when(i > 0)
            def _():
                pltpu.make_async_copy(buf_b, out_chunk(k0 - 1), sem_b).wait()
                gauss_scan(buf_b, k0 - 1, add=False)

            gauss_scan(buf_b, k0 + 1, add=True)

            @pl.when(k0 == NCHUNK - 2)
            def _():
                reg_scan(buf_b, (NCHUNK - 1) * CC, add=True)

            pltpu.make_async_copy(buf_b, out_chunk(k0 + 1), sem_b).start()
            return carry

        lax.fori_loop(0, NCHUNK // 2, pair, 0)
        pltpu.make_async_copy(buf_a, out_chunk(NCHUNK - 2), sem_a).wait()
        pltpu.make_async_copy(buf_b, out_chunk(NCHUNK - 1), sem_b).wait()

    return scatter_kernel(obj_t)


# speedup vs baseline: 8.8975x; 1.0259x over previous
import functools

import jax
import jax.numpy as jnp
from jax import lax
from jax.experimental import pallas as pl
from jax.experimental.pallas import tpu as pltpu
from jax.experimental.pallas import tpu_sc as plsc

BATCH = 32
NOBJ = 128
GRID = 128
NCH = 84
CC = 3
NCHUNK = NCH // CC
NLANE = 16
FARIDX = 1 << 27


def kernel(objects):
    obj_t = objects.reshape(BATCH, 6 * NOBJ)

    mesh = plsc.VectorSubcoreMesh(core_axis_name="c", subcore_axis_name="s")

    @functools.partial(
        pl.kernel,
        out_type=jax.ShapeDtypeStruct((BATCH, NCH, GRID, GRID), jnp.float32),
        mesh=mesh,
        compiler_params=pltpu.CompilerParams(needs_layout_passes=False),
        scratch_types=[
            pltpu.VMEM((6 * NOBJ,), jnp.float32),
            pltpu.VMEM((NOBJ * NLANE,), jnp.int32),
            pltpu.VMEM((NOBJ * NLANE,), jnp.float32),
            pltpu.VMEM((NOBJ * NLANE,), jnp.int32),
            pltpu.VMEM((NOBJ * NLANE,), jnp.float32),
            pltpu.VMEM((NCHUNK * NOBJ,), jnp.int32),
            pltpu.VMEM((32,), jnp.int32),
            pltpu.VMEM((CC, GRID, GRID), jnp.float32),
            pltpu.VMEM((CC, GRID, GRID), jnp.float32),
            pltpu.SemaphoreType.DMA,
            pltpu.SemaphoreType.DMA,
        ],
    )
    def scatter_kernel(obj_hbm, out_hbm, obj_v, gi_v, gv_v, ri_v, rv_v,
                       bkt_v, cnt_v, buf_a, buf_b, sem_a, sem_b):
        b = lax.axis_index("c") * 16 + lax.axis_index("s")
        pltpu.sync_copy(obj_hbm.at[b], obj_v)

        lane = lax.iota(jnp.int32, 16)
        is_tap = lane < 9
        is_reg = (lane >= 9) & (lane < 13)
        third = jnp.where(lane >= 3, 1, 0) + jnp.where(lane >= 6, 1, 0)
        dy = jnp.where(is_tap, third - 1, 0)
        dx = jnp.where(is_tap, lane - 3 * third - 1, 0)
        w_tap = jnp.exp(-1.125 * (dy * dy + dx * dx).astype(jnp.float32))
        reg_ch = 80 + (lane - 9)
        zeros_f = jnp.zeros((NLANE,), jnp.float32)
        lane0 = lane == 0

        cnt_v[pl.ds(0, NLANE)] = jnp.zeros((NLANE,), jnp.int32)
        cnt_v[pl.ds(NLANE, NLANE)] = jnp.zeros((NLANE,), jnp.int32)

        def build(n, carry):
            nv = jnp.full((NLANE,), n, jnp.int32)
            n6 = nv * 6
            y = plsc.load_gather(obj_v, [n6])
            x = plsc.load_gather(obj_v, [n6 + 1])
            hh = plsc.load_gather(obj_v, [n6 + 2])
            ww = plsc.load_gather(obj_v, [n6 + 3])
            cl = plsc.load_gather(obj_v, [n6 + 4])
            cf = plsc.load_gather(obj_v, [n6 + 5])
            yi = y.astype(jnp.int32)
            xi = x.astype(jnp.int32)
            ci = cl.astype(jnp.int32)
            oy = y - yi.astype(jnp.float32)
            ox = x - xi.astype(jnp.float32)
            ch = jnp.where(is_tap, ci, reg_ch)
            yy = yi + dy
            xx = xi + dx
            real = cf == 1.0
            inb = (yy >= 0) & (yy < GRID) & (xx >= 0) & (xx < GRID)
            m = real & inb
            idx = (ch << 14) + (yy << 7) + xx
            gi_v[pl.ds(n * NLANE, NLANE)] = jnp.where(
                m & is_tap, idx, FARIDX)
            gv_v[pl.ds(n * NLANE, NLANE)] = w_tap
            ri_v[pl.ds(n * NLANE, NLANE)] = jnp.where(
                m & is_reg, idx, FARIDX)
            val = jnp.where(lane == 9, oy, 0.0)
            val = jnp.where(lane == 10, ox, val)
            val = jnp.where(lane == 11, hh, val)
            val = jnp.where(lane == 12, ww, val)
            rv_v[pl.ds(n * NLANE, NLANE)] = val
            cid = (ci * 21846) >> 16
            cnt = plsc.load_gather(cnt_v, [cid])
            mask0 = lane0 & real
            plsc.store_scatter(bkt_v, [cid * NOBJ + cnt], nv, mask=mask0)
            plsc.store_scatter(cnt_v, [cid], cnt + 1, mask=mask0)
            return carry

        lax.fori_loop(0, NOBJ, build, 0)

        def zero_bufs(yrow, carry):
            for c in range(CC):
                for xb in range(GRID // NLANE):
                    buf_a[c, yrow, pl.ds(xb * NLANE, NLANE)] = zeros_f
                    buf_b[c, yrow, pl.ds(xb * NLANE, NLANE)] = zeros_f
            return carry

        lax.fori_loop(0, GRID, zero_bufs, 0)

        def gauss_scan(buf, k, add):
            kv = jnp.full((NLANE,), k, jnp.int32)
            cntk = jnp.max(plsc.load_gather(cnt_v, [kv]))
            c0 = k * CC

            def gbody(j, carry):
                nvec = plsc.load_gather(bkt_v, [kv * NOBJ + j])
                iv = plsc.load_gather(gi_v, [nvec * NLANE + lane])
                cv = (iv >> 14) - c0
                yv = (iv >> 7) & (GRID - 1)
                xv = iv & (GRID - 1)
                m = (cv >= 0) & (cv < CC)
                if add:
                    vv = plsc.load_gather(gv_v, [nvec * NLANE + lane])
                    plsc.addupdate_scatter(buf, [cv, yv, xv], vv, mask=m)
                else:
                    plsc.store_scatter(buf, [cv, yv, xv], zeros_f, mask=m)
                return carry

            lax.fori_loop(0, cntk, gbody, 0)

        def reg_scan(buf, c0, add):
            def rbody(n, carry):
                for t in range(4):
                    base = (n * 4 + t) * NLANE
                    iv = ri_v[pl.ds(base, NLANE)]
                    cv = (iv >> 14) - c0
                    yv = (iv >> 7) & (GRID - 1)
                    xv = iv & (GRID - 1)
                    m = (cv >= 0) & (cv < CC)
                    if add:
                        vv = rv_v[pl.ds(base, NLANE)]
                        plsc.addupdate_scatter(buf, [cv, yv, xv], vv, mask=m)
                    else:
                        plsc.store_scatter(buf, [cv, yv, xv], zeros_f, mask=m)
                return carry

            lax.fori_loop(0, NOBJ // 4, rbody, 0)

        def out_chunk(k):
            return out_hbm.at[b, pl.ds(k * CC, CC)]

        def pair(i, carry):
            k0 = 2 * i

            @pl.when(i > 0)
            def _():
                pltpu.make_async_copy(buf_a, out_chunk(k0 - 2), sem_a).wait()
                gauss_scan(buf_a, k0 - 2, add=False)

            gauss_scan(buf_a, k0, add=True)

            @pl.when(k0 == NCHUNK - 2)
            def _():
                reg_scan(buf_a, (NCHUNK - 2) * CC, add=True)

            pltpu.make_async_copy(buf_a, out_chunk(k0), sem_a).start()

            @pl.when(i > 0)
            def _():
                pltpu.make_async_copy(buf_b, out_chunk(k0 - 1), sem_b).wait()
                gauss_scan(buf_b, k0 - 1, add=False)

            gauss_scan(buf_b, k0 + 1, add=True)

            @pl.when(k0 == NCHUNK - 2)
            def _():
                reg_scan(buf_b, (NCHUNK - 1) * CC, add=True)

            pltpu.make_async_copy(buf_b, out_chunk(k0 + 1), sem_b).start()
            return carry

        lax.fori_loop(0, NCHUNK // 2, pair, 0)
        pltpu.make_async_copy(buf_a, out_chunk(NCHUNK - 2), sem_a).wait()
        pltpu.make_async_copy(buf_b, out_chunk(NCHUNK - 1), sem_b).wait()

    return scatter_kernel(obj_t)
